# baseline (speedup 1.0000x reference)
import functools

import jax
import jax.numpy as jnp
from jax import lax
from jax.experimental import pallas as pl
from jax.experimental.pallas import tpu as pltpu
from jax.experimental.pallas import tpu_sc as plsc

_NUM_ANCHORS = 9
_PRE = 6000
_POST = 300
_THRESH = 0.7
_B = 128
_NB = 48
_NPAD = _NB * _B
_SEL = 320
_NTR = 20736


def _nms_kernel(x1_ref, y1_ref, x2_ref, y2_ref, area_ref, pos_ref, keep_ref):
  f32 = jnp.float32
  ri = lax.broadcasted_iota(jnp.int32, (_NB, _B), 0)
  ci = lax.broadcasted_iota(jnp.int32, (_NB, _B), 1)
  keep_ref[...] = jnp.where(ri * _B + ci < _PRE, 1.0, 0.0).astype(f32)

  rr = lax.broadcasted_iota(jnp.int32, (_B, _B), 0)
  cc = lax.broadcasted_iota(jnp.int32, (_B, _B), 1)
  ident = (rr == cc).astype(f32)
  lower = (rr < cc).astype(f32)

  def _t(row):
    return lax.dot_general(ident, row, (((1,), (1,)), ((), ())),
                           preferred_element_type=f32)

  def _ov_mask(c_x1, c_y1, c_x2, c_y2, c_a, r_x1, r_y1, r_x2, r_y2, r_a):
    xx1 = jnp.maximum(c_x1, r_x1)
    yy1 = jnp.maximum(c_y1, r_y1)
    xx2 = jnp.minimum(c_x2, r_x2)
    yy2 = jnp.minimum(c_y2, r_y2)
    w = jnp.maximum(0.0, xx2 - xx1 + 1.0)
    h = jnp.maximum(0.0, yy2 - yy1 + 1.0)
    inter = w * h
    iou = inter / (c_a + r_a - inter)
    return (iou > _THRESH).astype(f32)

  def outer(i, carry):
    x1i = x1_ref[pl.ds(i, 1), :]
    y1i = y1_ref[pl.ds(i, 1), :]
    x2i = x2_ref[pl.ds(i, 1), :]
    y2i = y2_ref[pl.ds(i, 1), :]
    ai = area_ref[pl.ds(i, 1), :]
    cx1, cy1, cx2, cy2, ca = _t(x1i), _t(y1i), _t(x2i), _t(y2i), _t(ai)

    prior = keep_ref[pl.ds(i, 1), :]
    m_self = _ov_mask(cx1, cy1, cx2, cy2, ca, x1i, y1i, x2i, y2i, ai) * lower

    def fix_cond(c):
      return c[1] > 0

    def fix_body(c):
      a, _ = c
      dead = lax.dot_general(a, m_self, (((1,), (0,)), ((), ())),
                             preferred_element_type=f32)
      a_new = prior * jnp.where(dead > 0.0, 0.0, 1.0)
      changed = jnp.any(a_new != a).astype(jnp.int32)
      return (a_new, changed)

    a_fix, _ = lax.while_loop(fix_cond, fix_body, (prior, jnp.int32(1)))
    keep_ref[pl.ds(i, 1), :] = a_fix

    def inner(j, carry2):
      r_x1 = x1_ref[pl.ds(j, 1), :]
      r_y1 = y1_ref[pl.ds(j, 1), :]
      r_x2 = x2_ref[pl.ds(j, 1), :]
      r_y2 = y2_ref[pl.ds(j, 1), :]
      r_a = area_ref[pl.ds(j, 1), :]
      m = _ov_mask(cx1, cy1, cx2, cy2, ca, r_x1, r_y1, r_x2, r_y2, r_a)
      contrib = lax.dot_general(a_fix, m, (((1,), (0,)), ((), ())),
                                preferred_element_type=f32)
      keep_ref[pl.ds(j, 1), :] = (
          keep_ref[pl.ds(j, 1), :] * jnp.where(contrib > 0.0, 0.0, 1.0))
      return carry2

    lax.fori_loop(i + 1, _NB, inner, 0)
    return carry

  lax.fori_loop(0, _NB, outer, 0)

  keep = keep_ref[...]
  incl = (rr <= cc).astype(f32)
  lane_cum = lax.dot_general(keep, incl, (((1,), (0,)), ((), ())),
                             preferred_element_type=f32)
  ones_col = jnp.ones((_B, 1), f32)
  rowsum = lax.dot_general(keep, ones_col, (((1,), (0,)), ((), ())),
                           preferred_element_type=f32)
  r48 = lax.broadcasted_iota(jnp.int32, (_NB, _NB), 0)
  c48 = lax.broadcasted_iota(jnp.int32, (_NB, _NB), 1)
  l48 = (c48 < r48).astype(f32)
  row_off = lax.dot_general(l48, rowsum, (((1,), (0,)), ((), ())),
                            preferred_element_type=f32)
  ck = lane_cum + row_off
  ones_row = jnp.ones((1, _NB), f32)
  total = lax.dot_general(ones_row, rowsum, (((1,), (0,)), ((), ())),
                          preferred_element_type=f32)
  flat = (ri * _B + ci).astype(f32)
  pos = jnp.where(keep > 0.5, ck - 1.0, total + flat - ck)
  pos_ref[...] = pos.astype(jnp.int32)


def _nms_pos(x1, y1, x2, y2, area):
  return pl.pallas_call(
      _nms_kernel,
      out_shape=jax.ShapeDtypeStruct((_NB, _B), jnp.int32),
      scratch_shapes=[pltpu.VMEM((_NB, _B), jnp.float32)],
  )(x1, y1, x2, y2, area)


_NCH = 3


def _select_kernel(pos_hbm, iota_hbm, order_hbm, scores_hbm,
                   p0_hbm, p1_hbm, p2_hbm, p3_hbm,
                   t0_hbm, t1_hbm, t2_hbm, t3_hbm, t4_hbm, t5_hbm,
                   scores_o, props_o, trans_o,
                   pos_v, iota_v, sel_sh, sel_v, so_v, i2_v, po_v, to_v, sem):
  wid = lax.axis_index("s") * 2 + lax.axis_index("c")

  @pl.when(wid == 0)
  def _():
    pltpu.sync_copy(pos_hbm, pos_v)
    pltpu.sync_copy(iota_hbm, iota_v)

    descs = []
    for j in range(_NB):
      descs.append(
          pltpu.async_copy(iota_v.at[j], sel_sh.at[pos_v.at[j]], sem))
    for d in descs:
      d.wait()
    for j in range(_NCH):
      pltpu.sync_copy(sel_sh.at[pl.ds(j * 128, 128)], sel_v.at[j])

    pcols = (p0_hbm, p1_hbm, p2_hbm, p3_hbm)
    descs = []
    for j in range(_NCH):
      idx = sel_v.at[j]
      descs.append(pltpu.async_copy(scores_hbm.at[idx], so_v.at[j], sem))
      descs.append(pltpu.async_copy(order_hbm.at[idx], i2_v.at[j], sem))
      for c in range(4):
        descs.append(pltpu.async_copy(pcols[c].at[idx], po_v.at[c, j], sem))
    for d in descs:
      d.wait()

    tcols = (t0_hbm, t1_hbm, t2_hbm, t3_hbm, t4_hbm, t5_hbm)
    descs = []
    for j in range(_NCH):
      for c in range(6):
        descs.append(
            pltpu.async_copy(tcols[c].at[i2_v.at[j]], to_v.at[c, j], sem))
    for d in descs:
      d.wait()

    pltpu.sync_copy(so_v, scores_o)
    pltpu.sync_copy(po_v, props_o)
    pltpu.sync_copy(to_v, trans_o)


def _select(pos, iota, order, scores, pcols, tcols):
  f32, i32 = jnp.float32, jnp.int32
  mesh = plsc.VectorSubcoreMesh(core_axis_name="c", subcore_axis_name="s")
  k = functools.partial(
      pl.kernel,
      mesh=mesh,
      compiler_params=pltpu.CompilerParams(use_tc_tiling_on_sc=False),
      out_type=[
          jax.ShapeDtypeStruct((_NCH, 128), f32),
          jax.ShapeDtypeStruct((4, _NCH, 128), f32),
          jax.ShapeDtypeStruct((6, _NCH, 128), f32),
      ],
      scratch_types=[
          pltpu.VMEM((_NB, _B), i32),
          pltpu.VMEM((_NB, _B), i32),
          pltpu.VMEM_SHARED((_NPAD,), i32),
          pltpu.VMEM((_NCH, 128), i32),
          pltpu.VMEM((_NCH, 128), f32),
          pltpu.VMEM((_NCH, 128), i32),
          pltpu.VMEM((4, _NCH, 128), f32),
          pltpu.VMEM((6, _NCH, 128), f32),
          pltpu.SemaphoreType.DMA,
      ],
  )(_select_kernel)
  return k(pos, iota, order, scores, *pcols, *tcols)


def kernel(anchors, rpn_cls_prob, rpn_bbox_pred, rpn_trans_param, im_info):
  f32, i32 = jnp.float32, jnp.int32
  scores = rpn_cls_prob[0, :, :, _NUM_ANCHORS:].reshape(-1)
  deltas = rpn_bbox_pred.reshape(-1, 4)

  scores_sorted, order = lax.top_k(scores, _PRE)
  anch = jnp.take(anchors, order, axis=0)
  dels = jnp.take(deltas, order, axis=0)

  widths = anch[:, 2] - anch[:, 0] + 1.0
  heights = anch[:, 3] - anch[:, 1] + 1.0
  ctr_x = anch[:, 0] + 0.5 * widths
  ctr_y = anch[:, 1] + 0.5 * heights
  pred_ctr_x = dels[:, 0] * widths + ctr_x
  pred_ctr_y = dels[:, 1] * heights + ctr_y
  pred_w = jnp.exp(dels[:, 2]) * widths
  pred_h = jnp.exp(dels[:, 3]) * heights
  x1 = jnp.clip(pred_ctr_x - 0.5 * pred_w, 0.0, im_info[1] - 1.0)
  y1 = jnp.clip(pred_ctr_y - 0.5 * pred_h, 0.0, im_info[0] - 1.0)
  x2 = jnp.clip(pred_ctr_x + 0.5 * pred_w, 0.0, im_info[1] - 1.0)
  y2 = jnp.clip(pred_ctr_y + 0.5 * pred_h, 0.0, im_info[0] - 1.0)

  pad = _NPAD - _PRE
  padv = jnp.zeros((pad,), f32)
  x1p = jnp.concatenate([x1, padv]).reshape(_NB, _B)
  y1p = jnp.concatenate([y1, padv]).reshape(_NB, _B)
  x2p = jnp.concatenate([x2, padv]).reshape(_NB, _B)
  y2p = jnp.concatenate([y2, padv]).reshape(_NB, _B)
  areap = (x2p - x1p + 1.0) * (y2p - y1p + 1.0)

  pos = _nms_pos(x1p, y1p, x2p, y2p, areap)

  pcols = (x1p.reshape(-1), y1p.reshape(-1), x2p.reshape(-1), y2p.reshape(-1))
  iota = jnp.arange(_NPAD, dtype=i32).reshape(_NB, _B)
  order_pad = jnp.concatenate([order, jnp.zeros((pad,), i32)])
  scores_pad = jnp.concatenate([scores_sorted, padv])
  trans2d = rpn_trans_param.reshape(-1, 6)
  tcols = tuple(trans2d[:, c] for c in range(6))

  scores_o, props_o, trans_o = _select(pos, iota, order_pad, scores_pad,
                                       pcols, tcols)

  props_k = props_o.reshape(4, _NCH * 128).T[:_POST]
  scores_k = scores_o.reshape(-1)[:_POST]
  trans_k = trans_o.reshape(6, _NCH * 128).T[:_POST]
  blob = jnp.concatenate([jnp.zeros((_POST, 1), f32), props_k], axis=1)
  return (blob, scores_k, trans_k)
